# baseline (speedup 1.0000x reference)
import functools

import jax
import jax.numpy as jnp
from jax import lax
from jax.experimental import pallas as pl
from jax.experimental.pallas import tpu as pltpu
from jax.experimental.pallas import tpu_sc as plsc

B = 1024
N = 200
D = 64
V = 100000
L = 16
NC = 2
NS = 16
NW = NC * NS
D_PER_W = D // NW
RB = 8
CB = 256
NBLK_C = B // CB
NBLK = (N // RB) * NBLK_C

_mesh = plsc.VectorSubcoreMesh(core_axis_name="c", subcore_axis_name="s")

_cp = pltpu.CompilerParams(
    needs_layout_passes=False,
    use_tc_tiling_on_sc=True,
)


@functools.partial(
    pl.kernel,
    mesh=_mesh,
    compiler_params=_cp,
    out_type=jax.ShapeDtypeStruct((D, 2, N, B), jnp.float32),
    scratch_types=[
        pltpu.VMEM((V,), jnp.float32),
        pltpu.VMEM((RB, CB), jnp.int32),
        pltpu.VMEM((RB, CB), jnp.int32),
        pltpu.VMEM((RB, CB), jnp.float32),
        pltpu.VMEM((RB, CB), jnp.float32),
        pltpu.VMEM((RB, B), jnp.float32),
        pltpu.SemaphoreType.DMA,
        pltpu.SemaphoreType.DMA,
        pltpu.SemaphoreType.DMA,
        pltpu.SemaphoreType.DMA,
        pltpu.SemaphoreType.DMA,
        pltpu.SemaphoreType.DMA,
        pltpu.VMEM_SHARED((N, B), jnp.int32),
        pltpu.SemaphoreType.DMA,
    ],
)
def _sc_embed(idx_hbm, table_hbm, out_hbm, trow, iblk0, iblk1,
              oblk0, oblk1, zblk, isem0, isem1, osem0, osem1, zsem, tsem,
              idx_sh, lsem):
    sid = lax.axis_index("s")
    wid = sid * NC + lax.axis_index("c")

    zeros16 = jnp.zeros((L,), jnp.float32)

    @pl.loop(0, RB)
    def _(r):
        @pl.loop(0, B, step=L)
        def _(c):
            zblk[r, pl.ds(c, L)] = zeros16

    def blk_pos(ii):
        r0 = (ii // NBLK_C) * RB
        c0 = (ii % NBLK_C) * CB
        return r0, c0

    def idx_src(ii):
        r0, c0 = blk_pos(ii)
        return idx_sh.at[pl.ds(r0, RB), pl.ds(c0, CB)]

    slots = ((iblk0, oblk0, isem0, osem0), (iblk1, oblk1, isem1, osem1))

    pltpu.async_copy(table_hbm.at[wid * D_PER_W], trow, tsem)

    @pl.when(sid == 0)
    def _():
        pltpu.async_copy(idx_hbm, idx_sh, lsem).wait()

    plsc.subcore_barrier()

    for dd in range(D_PER_W):
        d = wid * D_PER_W + dd

        pltpu.make_async_copy(table_hbm.at[0], trow, tsem).wait()

        pltpu.async_copy(idx_src(0), iblk0, isem0)
        pltpu.async_copy(idx_src(1), iblk1, isem1)

        @pl.loop(0, NBLK, step=2)
        def _(i):
            for s, (iblk, oblk, isem, osem) in enumerate(slots):
                ii = i + s
                r0, c0 = blk_pos(ii)
                dst = out_hbm.at[d, 0, pl.ds(r0, RB), pl.ds(c0, CB)]

                pltpu.make_async_copy(idx_src(0), iblk, isem).wait()

                @pl.when(ii >= 2)
                def _():
                    pltpu.make_async_copy(oblk, dst, osem).wait()

                @plsc.parallel_loop(0, CB, step=L, unroll=2)
                def _(c):
                    for r in range(RB):
                        iv = iblk[r, pl.ds(c, L)]
                        oblk[r, pl.ds(c, L)] = plsc.load_gather(trow, [iv])

                pltpu.async_copy(oblk, dst, osem)

                if s == 0:
                    zdst = out_hbm.at[d, 1, pl.ds(r0, RB), :]
                    first_col = (ii % NBLK_C) == 0

                    @pl.when(first_col & (ii >= NBLK_C))
                    def _():
                        pltpu.make_async_copy(zblk, zdst, zsem).wait()

                    @pl.when(first_col)
                    def _():
                        pltpu.async_copy(zblk, zdst, zsem)

                @pl.when(ii + 2 < NBLK)
                def _():
                    pltpu.async_copy(idx_src(ii + 2), iblk, isem)

        if dd + 1 < D_PER_W:
            pltpu.async_copy(table_hbm.at[d + 1], trow, tsem)

        pltpu.make_async_copy(
            oblk0, out_hbm.at[d, 0, pl.ds(0, RB), pl.ds(0, CB)], osem0).wait()
        pltpu.make_async_copy(
            oblk1, out_hbm.at[d, 0, pl.ds(0, RB), pl.ds(0, CB)], osem1).wait()
        pltpu.make_async_copy(
            zblk, out_hbm.at[d, 1, pl.ds(0, RB), :], zsem).wait()


def kernel(atomic_numbers, table):
    idx_t = atomic_numbers.T
    table_t = table.T
    out = _sc_embed(idx_t, table_t)
    return jnp.transpose(out, (3, 0, 1, 2)).reshape(B, D, 1, 2, 1, 1, N)


# speedup vs baseline: 1.0378x; 1.0378x over previous
import functools

import jax
import jax.numpy as jnp
from jax import lax
from jax.experimental import pallas as pl
from jax.experimental.pallas import tpu as pltpu
from jax.experimental.pallas import tpu_sc as plsc

B = 1024
N = 200
D = 64
V = 100000
L = 16
NC = 2
NS = 16
NW = NC * NS
D_PER_W = D // NW
RB = 8
CB = 256
NBLK_C = B // CB
NBLK = (N // RB) * NBLK_C

_mesh = plsc.VectorSubcoreMesh(core_axis_name="c", subcore_axis_name="s")

_cp = pltpu.CompilerParams(
    needs_layout_passes=False,
    use_tc_tiling_on_sc=True,
)


@functools.partial(
    pl.kernel,
    mesh=_mesh,
    compiler_params=_cp,
    out_type=jax.ShapeDtypeStruct((D, 2, N, B), jnp.float32),
    scratch_types=[
        pltpu.VMEM((V,), jnp.float32),
        pltpu.VMEM((RB, CB), jnp.int32),
        pltpu.VMEM((RB, CB), jnp.int32),
        pltpu.VMEM((RB, CB), jnp.float32),
        pltpu.VMEM((RB, CB), jnp.float32),
        pltpu.VMEM((RB, B), jnp.float32),
        pltpu.SemaphoreType.DMA,
        pltpu.SemaphoreType.DMA,
        pltpu.SemaphoreType.DMA,
        pltpu.SemaphoreType.DMA,
        pltpu.SemaphoreType.DMA,
        pltpu.SemaphoreType.DMA,
        pltpu.VMEM_SHARED((N, B), jnp.int32),
        pltpu.SemaphoreType.DMA,
    ],
)
def _sc_embed(idx_hbm, table_hbm, out_hbm, trow, iblk0, iblk1,
              oblk0, oblk1, zblk, isem0, isem1, osem0, osem1, zsem, tsem,
              idx_sh, lsem):
    sid = lax.axis_index("s")
    wid = sid * NC + lax.axis_index("c")

    zeros16 = jnp.zeros((L,), jnp.float32)

    @pl.loop(0, RB)
    def _(r):
        @pl.loop(0, B, step=L)
        def _(c):
            zblk[r, pl.ds(c, L)] = zeros16

    def blk_pos(ii):
        r0 = (ii // NBLK_C) * RB
        c0 = (ii % NBLK_C) * CB
        return r0, c0

    def idx_src(ii):
        r0, c0 = blk_pos(ii)
        return idx_sh.at[pl.ds(r0, RB), pl.ds(c0, CB)]

    slots = ((iblk0, oblk0, isem0, osem0), (iblk1, oblk1, isem1, osem1))

    pltpu.async_copy(table_hbm.at[wid], trow, tsem)

    @pl.when(sid == 0)
    def _():
        pltpu.async_copy(idx_hbm, idx_sh, lsem).wait()

    plsc.subcore_barrier()

    for dd in range(D_PER_W):
        d = wid + NW * dd

        pltpu.make_async_copy(table_hbm.at[0], trow, tsem).wait()

        pltpu.async_copy(idx_src(0), iblk0, isem0)
        pltpu.async_copy(idx_src(1), iblk1, isem1)

        @pl.loop(0, NBLK, step=2)
        def _(i):
            for s, (iblk, oblk, isem, osem) in enumerate(slots):
                ii = i + s
                r0, c0 = blk_pos(ii)
                dst = out_hbm.at[d, 0, pl.ds(r0, RB), pl.ds(c0, CB)]

                pltpu.make_async_copy(idx_src(0), iblk, isem).wait()

                @pl.when(ii >= 2)
                def _():
                    pltpu.make_async_copy(oblk, dst, osem).wait()

                @plsc.parallel_loop(0, CB, step=L, unroll=4)
                def _(c):
                    for r in range(RB):
                        iv = iblk[r, pl.ds(c, L)]
                        oblk[r, pl.ds(c, L)] = plsc.load_gather(trow, [iv])

                pltpu.async_copy(oblk, dst, osem)

                if s == 0:
                    zdst = out_hbm.at[d, 1, pl.ds(r0, RB), :]
                    first_col = (ii % NBLK_C) == 0

                    @pl.when(first_col & (ii >= NBLK_C))
                    def _():
                        pltpu.make_async_copy(zblk, zdst, zsem).wait()

                    @pl.when(first_col)
                    def _():
                        pltpu.async_copy(zblk, zdst, zsem)

                @pl.when(ii + 2 < NBLK)
                def _():
                    pltpu.async_copy(idx_src(ii + 2), iblk, isem)

        if dd + 1 < D_PER_W:
            pltpu.async_copy(table_hbm.at[d + NW], trow, tsem)

        pltpu.make_async_copy(
            oblk0, out_hbm.at[d, 0, pl.ds(0, RB), pl.ds(0, CB)], osem0).wait()
        pltpu.make_async_copy(
            oblk1, out_hbm.at[d, 0, pl.ds(0, RB), pl.ds(0, CB)], osem1).wait()
        pltpu.make_async_copy(
            zblk, out_hbm.at[d, 1, pl.ds(0, RB), :], zsem).wait()


def kernel(atomic_numbers, table):
    idx_t = atomic_numbers.T
    table_t = table.T
    out = _sc_embed(idx_t, table_t)
    return jnp.transpose(out, (3, 0, 1, 2)).reshape(B, D, 1, 2, 1, 1, N)
